# Initial kernel scaffold; baseline (speedup 1.0000x reference)
import jax
import jax.numpy as jnp
from jax.experimental import pallas as pl


def kernel(i, table):
    raise NotImplementedError("write your pallas kernel here")


# speedup vs baseline: 4.8060x; 4.8060x over previous
import functools

import jax
import jax.numpy as jnp
from jax import lax
from jax.experimental import pallas as pl
from jax.experimental.pallas import tpu as pltpu
from jax.experimental.pallas import tpu_sc as plsc

NUM_WORKERS = 32
CHUNK = 1024


@functools.lru_cache(maxsize=None)
def _build(n_total, vocab, dim):
  per_w = n_total // NUM_WORKERS
  n_chunks = per_w // CHUNK
  mesh = plsc.VectorSubcoreMesh(core_axis_name="c", subcore_axis_name="s")

  @functools.partial(
      pl.kernel,
      mesh=mesh,
      out_type=jax.ShapeDtypeStruct((n_total, dim), jnp.float32),
      compiler_params=pltpu.CompilerParams(use_tc_tiling_on_sc=False),
      scratch_types=[
          pltpu.VMEM((CHUNK,), jnp.int32),
          pltpu.VMEM((CHUNK, dim), jnp.float32),
          pltpu.SemaphoreType.DMA,
      ],
  )
  def emb(idx_hbm, table_hbm, out_hbm, idx_v, rows_v, gsem):
    wid = lax.axis_index("s") * 2 + lax.axis_index("c")
    base = wid * per_w

    def body(g, carry):
      start = base + g * CHUNK
      pltpu.sync_copy(idx_hbm.at[pl.ds(start, CHUNK)], idx_v)
      pltpu.async_copy(table_hbm.at[idx_v], rows_v, gsem).wait()
      pltpu.sync_copy(rows_v, out_hbm.at[pl.ds(start, CHUNK)])
      return carry

    lax.fori_loop(0, n_chunks, body, 0)

  return emb


def kernel(i, table):
  b, h = i.shape
  vocab, dim = table.shape
  n_total = b * h
  out = _build(n_total, vocab, dim)(i.reshape(n_total), table)
  return out.reshape(b, h, dim)


# speedup vs baseline: 5.0517x; 1.0511x over previous
import functools

import jax
import jax.numpy as jnp
from jax import lax
from jax.experimental import pallas as pl
from jax.experimental.pallas import tpu as pltpu
from jax.experimental.pallas import tpu_sc as plsc

NUM_WORKERS = 32
CHUNK = 800
NB = 4
K = 2


@functools.lru_cache(maxsize=None)
def _build(n_total, vocab, dim):
  per_w = n_total // NUM_WORKERS
  assert per_w * NUM_WORKERS == n_total
  n = per_w // CHUNK
  assert n * CHUNK == per_w and n % NB == 0 and n // NB >= 2
  mesh = plsc.VectorSubcoreMesh(core_axis_name="c", subcore_axis_name="s")

  @functools.partial(
      pl.kernel,
      mesh=mesh,
      out_type=jax.ShapeDtypeStruct((n_total, dim), jnp.float32),
      compiler_params=pltpu.CompilerParams(use_tc_tiling_on_sc=False),
      scratch_types=(
          [pltpu.VMEM((NB, CHUNK), jnp.int32),
           pltpu.VMEM((NB, CHUNK, dim), jnp.float32)]
          + [pltpu.SemaphoreType.DMA] * (3 * NB)
      ),
  )
  def emb(idx_hbm, table_hbm, out_hbm, idx_v, rows_v, *sems):
    lsem = sems[0:NB]
    gsem = sems[NB:2 * NB]
    ssem = sems[2 * NB:3 * NB]
    wid = lax.axis_index("s") * 2 + lax.axis_index("c")
    base = wid * per_w

    def idx_load(g, b):
      return pltpu.make_async_copy(
          idx_hbm.at[pl.ds(base + g * CHUNK, CHUNK)], idx_v.at[b], lsem[b])

    def gath(b):
      return pltpu.make_async_copy(
          table_hbm.at[idx_v.at[b]], rows_v.at[b], gsem[b])

    def store(g, b):
      return pltpu.make_async_copy(
          rows_v.at[b], out_hbm.at[pl.ds(base + g * CHUNK, CHUNK)], ssem[b])

    for b in range(NB):
      idx_load(b, b).start()
    for j in range(K):
      idx_load(j, j).wait()
      gath(j).start()

    def step(g, b, do_idx_load, do_store_wait):
      gath(b).wait()
      store(g, b).start()
      if do_idx_load:
        idx_load(g + NB, b).start()
      b2 = (b + K) % NB
      if do_store_wait:
        store(g + K - NB, b2).wait()
      idx_load(g + K, b2).wait()
      gath(b2).start()

    for b in range(NB):
      step(b, b, True, b + K >= NB)

    def outer(go, carry):
      for b in range(NB):
        step(go * NB + b, b, True, True)
      return carry

    lax.fori_loop(1, n // NB - 1, outer, 0)

    for b in range(NB):
      g = n - NB + b
      gath(b).wait()
      store(g, b).start()
      if g + K < n:
        b2 = (b + K) % NB
        store(g + K - NB, b2).wait()
        idx_load(g + K, b2).wait()
        gath(b2).start()
    for b in range(NB):
      store(n - NB + b, b).wait()

  return emb


def kernel(i, table):
  b, h = i.shape
  vocab, dim = table.shape
  n_total = b * h
  out = _build(n_total, vocab, dim)(i.reshape(n_total), table)
  return out.reshape(b, h, dim)
